# baseline (speedup 1.0000x reference)
import functools

import numpy as np
import jax
import jax.numpy as jnp
from jax import lax
from jax.experimental import pallas as pl
from jax.experimental.pallas import tpu as pltpu
from jax.experimental.pallas import tpu_sc as plsc

_D_MODEL = 64
_CONTEXT_LEN = 200


def _sinusoidal_table(d_model: int, context_len: int) -> np.ndarray:
    pos = np.arange(context_len, dtype=np.float32)[:, None]
    i = np.arange(d_model, dtype=np.float32)[None, :]
    angle_rates = 1.0 / np.power(10000.0, (2.0 * np.floor(i / 2.0)) / float(d_model))
    angles = pos * angle_rates
    table = np.zeros((context_len, d_model), dtype=np.float32)
    table[:, 0::2] = np.sin(angles[:, 0::2])
    table[:, 1::2] = np.cos(angles[:, 1::2])
    return table


def kernel(text_batch, embedding_matrix):
    batch, seq_len = text_batch.shape
    vocab, d_model = embedding_matrix.shape
    assert seq_len == _CONTEXT_LEN and d_model == _D_MODEL

    n_tokens = batch * seq_len
    flat_idx = text_batch.reshape(n_tokens)

    info = plsc.get_sparse_core_info()
    num_workers = info.num_cores * info.num_subcores
    per_worker = n_tokens // num_workers
    assert per_worker * num_workers == n_tokens
    chunk = seq_len
    n_chunks = per_worker // chunk
    assert n_chunks * chunk == per_worker

    pos_table = jnp.asarray(_sinusoidal_table(d_model, seq_len))

    mesh = plsc.VectorSubcoreMesh(core_axis_name="c", subcore_axis_name="s")

    @functools.partial(
        pl.kernel,
        mesh=mesh,
        out_type=jax.ShapeDtypeStruct((n_tokens, d_model), jnp.float32),
        scratch_types=[
            pltpu.VMEM((per_worker,), jnp.int32),
            pltpu.VMEM((chunk, d_model), jnp.float32),
            pltpu.VMEM((seq_len, d_model), jnp.float32),
            pltpu.SemaphoreType.DMA,
        ],
        compiler_params=pltpu.CompilerParams(use_tc_tiling_on_sc=False),
    )
    def _emb_kernel(idx_hbm, table_hbm, pos_hbm, out_hbm, idx_v, rows_v, pos_v, sem):
        wid = lax.axis_index("s") * info.num_cores + lax.axis_index("c")
        base = wid * per_worker
        pltpu.sync_copy(idx_hbm.at[pl.ds(base, per_worker)], idx_v)
        pltpu.sync_copy(pos_hbm, pos_v)

        def chunk_body(j, carry):
            off = j * chunk
            pltpu.async_copy(table_hbm.at[idx_v.at[pl.ds(off, chunk)]], rows_v, sem).wait()

            def row_body(r, c2):
                for c in range(d_model // 16):
                    sl = pl.ds(c * 16, 16)
                    rows_v[r, sl] = rows_v[r, sl] + pos_v[r, sl]
                return c2

            lax.fori_loop(0, chunk, row_body, 0)
            pltpu.sync_copy(rows_v, out_hbm.at[pl.ds(base + off, chunk)])
            return carry

        lax.fori_loop(0, n_chunks, chunk_body, 0)

    out = _emb_kernel(flat_idx, embedding_matrix, pos_table)
    return out.reshape(batch, seq_len, d_model)
